# baseline (speedup 1.0000x reference)
import math

import jax
import jax.numpy as jnp
from jax import lax
from jax.experimental import pallas as pl
from jax.experimental.pallas import tpu as pltpu
from jax.experimental.pallas import tpu_sc as plsc

D = 128
H = 4
DH = D // H
N = 10000
E = 320000
NC = 2
NS = 16
NW = NC * NS
C = 128
EPT_A = E // NW
NCH_A = EPT_A // C
TAIL_A = EPT_A - NCH_A * C
EPT_B = E // NS
NCH_B = EPT_B // C
TAIL_B = EPT_B - NCH_B * C
RB = 624
RTAIL = N - RB * NS
L = 16

_f32 = jnp.float32
_i32 = jnp.int32


def _qkv_body(x_ref, wq_ref, bq_ref, wk_ref, bk_ref, wv_ref, bv_ref,
              q_ref, k_ref, v_ref):
    x = x_ref[...]
    q_ref[...] = jnp.dot(x, wq_ref[...], preferred_element_type=_f32) + bq_ref[...]
    k = jnp.dot(x, wk_ref[...], preferred_element_type=_f32) + bk_ref[...]
    k_ref[...] = k * (1.0 / math.sqrt(DH))
    v_ref[...] = jnp.dot(x, wv_ref[...], preferred_element_type=_f32) + bv_ref[...]


def _qkv(x, wq, bq, wk, bk, wv, bv):
    blk = 1000
    row_spec = pl.BlockSpec((blk, D), lambda i: (i, 0))
    full_w = pl.BlockSpec((D, D), lambda i: (0, 0))
    full_b = pl.BlockSpec((1, D), lambda i: (0, 0))
    return pl.pallas_call(
        _qkv_body,
        grid=(N // blk,),
        in_specs=[row_spec, full_w, full_b, full_w, full_b, full_w, full_b],
        out_specs=[row_spec, row_spec, row_spec],
        out_shape=[jax.ShapeDtypeStruct((N, D), _f32)] * 3,
    )(x, wq, bq.reshape(1, D), wk, bk.reshape(1, D), wv, bv.reshape(1, D))


def _sc_gather_body(q_hbm, k_hbm, tgt_hbm, src_hbm,
                    qg_hbm, kg_hbm,
                    qbuf, kbuf, tgtbuf, srcbuf,
                    qbt, kbt, tgtbt, srcbt,
                    sem0, sem1):
    cid = lax.axis_index("c")
    sid = lax.axis_index("s")
    wid = sid * NC + cid
    tile_base = wid * EPT_A

    def _do_chunk(base, n, tb, sb, qb, kb):
        cp_t = pltpu.async_copy(tgt_hbm.at[pl.ds(base, n)], tb, sem0)
        cp_s = pltpu.async_copy(src_hbm.at[pl.ds(base, n)], sb, sem1)
        cp_t.wait()
        cp_s.wait()
        gq = pltpu.async_copy(q_hbm.at[tb], qb, sem0)
        gk = pltpu.async_copy(k_hbm.at[sb], kb, sem1)
        gq.wait()
        gk.wait()
        pltpu.sync_copy(qb, qg_hbm.at[pl.ds(base, n)])
        pltpu.sync_copy(kb, kg_hbm.at[pl.ds(base, n)])

    def _chunk(ci, carry):
        _do_chunk(tile_base + ci * C, C, tgtbuf, srcbuf, qbuf, kbuf)
        return carry

    lax.fori_loop(0, NCH_A, _chunk, 0)
    _do_chunk(tile_base + NCH_A * C, TAIL_A, tgtbt, srcbt, qbt, kbt)


def _sc_gather(q, k, tgt, src):
    mesh = plsc.VectorSubcoreMesh(
        core_axis_name="c", subcore_axis_name="s", num_cores=NC, num_subcores=NS)
    fn = pl.kernel(
        _sc_gather_body,
        out_type=(jax.ShapeDtypeStruct((E, D), _f32),
                  jax.ShapeDtypeStruct((E, D), _f32)),
        mesh=mesh,
        scratch_types=[
            pltpu.VMEM((C, D), _f32),
            pltpu.VMEM((C, D), _f32),
            pltpu.VMEM((C,), _i32),
            pltpu.VMEM((C,), _i32),
            pltpu.VMEM((TAIL_A, D), _f32),
            pltpu.VMEM((TAIL_A, D), _f32),
            pltpu.VMEM((TAIL_A,), _i32),
            pltpu.VMEM((TAIL_A,), _i32),
            pltpu.SemaphoreType.DMA,
            pltpu.SemaphoreType.DMA,
        ],
    )
    return fn(q, k, tgt, src)


def _edge_body(qg_ref, kg_ref, s_ref):
    p = qg_ref[...] * kg_ref[...]
    colh = lax.broadcasted_iota(_i32, (H, D), 1) // DH
    rowh = lax.broadcasted_iota(_i32, (H, D), 0)
    selm = (colh == rowh).astype(_f32)
    s4 = jnp.dot(p, selm.T, preferred_element_type=_f32)
    s_ref[...] = jnp.dot(jnp.exp(s4), selm, preferred_element_type=_f32)


def _edge(qg, kg):
    blk = 2000
    row_spec = pl.BlockSpec((blk, D), lambda i: (i, 0))
    return pl.pallas_call(
        _edge_body,
        grid=(E // blk,),
        in_specs=[row_spec, row_spec],
        out_specs=row_spec,
        out_shape=jax.ShapeDtypeStruct((E, D), _f32),
    )(qg, kg)


def _sc_scatter_body(v_hbm, s_hbm, tgt_hbm, src_hbm,
                     acc_hbm,
                     buf, ebuf, tgtbuf, srcbuf, buft, ebuft, tgtbt, srcbt,
                     sem0, sem1, sem2,
                     acc_sh):
    cid = lax.axis_index("c")
    sid = lax.axis_index("s")
    row0 = sid * RB
    is_last = sid == NS - 1
    tile_base = sid * EPT_B
    zeros16 = jnp.zeros((L,), _f32)

    def _zw(i, c):
        for j in range(D // L):
            buf[i, pl.ds(j * L, L)] = zeros16
        return c
    lax.fori_loop(0, C, _zw, 0)
    off = 0
    while off < RB:
        sz = min(C, RB - off)
        pltpu.sync_copy(buf.at[pl.ds(0, sz)], acc_sh.at[pl.ds(row0 + off, sz)])
        off += sz

    @pl.when(is_last)
    def _zero_tail():
        pltpu.sync_copy(buf.at[pl.ds(0, RTAIL)], acc_sh.at[pl.ds(RB * NS, RTAIL)])

    plsc.subcore_barrier()

    def _do_chunk_w(base, n, tb, sb, rb, eb):
        cp_t = pltpu.async_copy(tgt_hbm.at[pl.ds(base, n)], tb, sem0)
        cp_s = pltpu.async_copy(src_hbm.at[pl.ds(base, n)], sb, sem1)
        cp_e = pltpu.async_copy(s_hbm.at[pl.ds(base, n)], eb, sem2)
        cp_s.wait()
        gv = pltpu.async_copy(v_hbm.at[sb], rb, sem1)
        cp_t.wait()
        cp_e.wait()
        gv.wait()

        def _mul(e, c):
            for j in range(D // L):
                sl = pl.ds(j * L, L)
                rb[e, sl] = rb[e, sl] * eb[e, sl]
            return c
        lax.fori_loop(0, n, _mul, 0)
        pltpu.sync_copy(rb, acc_sh.at[tb], add=True)

    def _do_chunk_s(base, n, tb, eb):
        cp_t = pltpu.async_copy(tgt_hbm.at[pl.ds(base, n)], tb, sem0)
        cp_e = pltpu.async_copy(s_hbm.at[pl.ds(base, n)], eb, sem2)
        cp_t.wait()
        cp_e.wait()
        pltpu.sync_copy(eb, acc_sh.at[tb], add=True)

    @pl.when(cid == 0)
    def _do_w():
        def _chunk(ci, carry):
            _do_chunk_w(tile_base + ci * C, C, tgtbuf, srcbuf, buf, ebuf)
            return carry
        lax.fori_loop(0, NCH_B, _chunk, 0)
        _do_chunk_w(tile_base + NCH_B * C, TAIL_B, tgtbt, srcbt, buft, ebuft)

    @pl.when(cid == 1)
    def _do_s():
        def _chunk(ci, carry):
            _do_chunk_s(tile_base + ci * C, C, tgtbuf, ebuf)
            return carry
        lax.fori_loop(0, NCH_B, _chunk, 0)
        _do_chunk_s(tile_base + NCH_B * C, TAIL_B, tgtbt, ebuft)

    plsc.subcore_barrier()

    pltpu.sync_copy(acc_sh.at[pl.ds(row0, RB)], acc_hbm.at[cid, pl.ds(row0, RB)])

    @pl.when(is_last)
    def _write_tail():
        pltpu.sync_copy(acc_sh.at[pl.ds(RB * NS, RTAIL)],
                        acc_hbm.at[cid, pl.ds(RB * NS, RTAIL)])


def _sc_scatter(v, s, tgt, src):
    mesh = plsc.VectorSubcoreMesh(
        core_axis_name="c", subcore_axis_name="s", num_cores=NC, num_subcores=NS)
    fn = pl.kernel(
        _sc_scatter_body,
        out_type=jax.ShapeDtypeStruct((NC, N, D), _f32),
        mesh=mesh,
        scratch_types=[
            pltpu.VMEM((C, D), _f32),
            pltpu.VMEM((C, D), _f32),
            pltpu.VMEM((C,), _i32),
            pltpu.VMEM((C,), _i32),
            pltpu.VMEM((TAIL_B, D), _f32),
            pltpu.VMEM((TAIL_B, D), _f32),
            pltpu.VMEM((TAIL_B,), _i32),
            pltpu.VMEM((TAIL_B,), _i32),
            pltpu.SemaphoreType.DMA,
            pltpu.SemaphoreType.DMA,
            pltpu.SemaphoreType.DMA,
            pltpu.VMEM_SHARED((N, D), _f32),
        ],
    )
    return fn(v, s, tgt, src)


def _final_body(x_ref, acc_ref, wo_ref, bo_ref, g_ref, b_ref, o_ref):
    w = acc_ref[0] / (acc_ref[1] + 1e-10)
    y = jnp.dot(w, wo_ref[...], preferred_element_type=_f32) + bo_ref[...] + x_ref[...]
    mu = jnp.mean(y, axis=1, keepdims=True)
    dev = y - mu
    var = jnp.mean(dev * dev, axis=1, keepdims=True)
    o_ref[...] = g_ref[...] * (dev * lax.rsqrt(var + 1e-5)) + b_ref[...]


def _final(x, acc, wo, bo, gamma, beta):
    blk = 1000
    row_spec = pl.BlockSpec((blk, D), lambda i: (i, 0))
    return pl.pallas_call(
        _final_body,
        grid=(N // blk,),
        in_specs=[
            row_spec,
            pl.BlockSpec((NC, blk, D), lambda i: (0, i, 0)),
            pl.BlockSpec((D, D), lambda i: (0, 0)),
            pl.BlockSpec((1, D), lambda i: (0, 0)),
            pl.BlockSpec((1, D), lambda i: (0, 0)),
            pl.BlockSpec((1, D), lambda i: (0, 0)),
        ],
        out_specs=row_spec,
        out_shape=jax.ShapeDtypeStruct((N, D), _f32),
    )(x, acc, wo, bo.reshape(1, D), gamma.reshape(1, D), beta.reshape(1, D))


@jax.jit
def kernel(node_features, edge_index, Wq, bq, Wk, bk, Wv, bv, Wo, bo, gamma, beta):
    src = edge_index[0].astype(_i32)
    tgt = edge_index[1].astype(_i32)
    q, k, v = _qkv(node_features, Wq, bq, Wk, bk, Wv, bv)
    qg, kg = _sc_gather(q, k, tgt, src)
    s = _edge(qg, kg)
    acc = _sc_scatter(v, s, tgt, src)
    return _final(node_features, acc, Wo, bo, gamma, beta)


# speedup vs baseline: 1.0940x; 1.0940x over previous
import math

import jax
import jax.numpy as jnp
from jax import lax
from jax.experimental import pallas as pl
from jax.experimental.pallas import tpu as pltpu
from jax.experimental.pallas import tpu_sc as plsc

D = 128
H = 4
DH = D // H
N = 10000
E = 320000
NC = 2
NS = 16
NW = NC * NS
C = 128
EPT_A = E // NW
NCH_A = EPT_A // C
TAIL_A = EPT_A - NCH_A * C
EPT_B = E // NS
NCH_B = EPT_B // C
TAIL_B = EPT_B - NCH_B * C
RB = 624
RTAIL = N - RB * NS
L = 16

_f32 = jnp.float32
_i32 = jnp.int32


def _qkv_body(x_ref, wq_ref, bq_ref, wk_ref, bk_ref, wv_ref, bv_ref,
              q_ref, k_ref, v_ref):
    x = x_ref[...]
    q_ref[...] = jnp.dot(x, wq_ref[...], preferred_element_type=_f32) + bq_ref[...]
    k = jnp.dot(x, wk_ref[...], preferred_element_type=_f32) + bk_ref[...]
    k_ref[...] = k * (1.0 / math.sqrt(DH))
    v_ref[...] = jnp.dot(x, wv_ref[...], preferred_element_type=_f32) + bv_ref[...]


def _qkv(x, wq, bq, wk, bk, wv, bv):
    blk = 1000
    row_spec = pl.BlockSpec((blk, D), lambda i: (i, 0))
    full_w = pl.BlockSpec((D, D), lambda i: (0, 0))
    full_b = pl.BlockSpec((1, D), lambda i: (0, 0))
    return pl.pallas_call(
        _qkv_body,
        grid=(N // blk,),
        in_specs=[row_spec, full_w, full_b, full_w, full_b, full_w, full_b],
        out_specs=[row_spec, row_spec, row_spec],
        out_shape=[jax.ShapeDtypeStruct((N, D), _f32)] * 3,
    )(x, wq, bq.reshape(1, D), wk, bk.reshape(1, D), wv, bv.reshape(1, D))


def _sc_gather_body(q_hbm, k_hbm, v_hbm, tgt_hbm, src_hbm,
                    qg_hbm, kg_hbm, vg_hbm,
                    qbuf, kbuf, vbuf, tgtbuf, srcbuf,
                    qbt, kbt, vbt, tgtbt, srcbt,
                    sem0, sem1, sem2):
    cid = lax.axis_index("c")
    sid = lax.axis_index("s")
    wid = sid * NC + cid
    tile_base = wid * EPT_A

    def _do_chunk(base, n, tb, sb, qb, kb, vb):
        cp_t = pltpu.async_copy(tgt_hbm.at[pl.ds(base, n)], tb, sem0)
        cp_s = pltpu.async_copy(src_hbm.at[pl.ds(base, n)], sb, sem1)
        cp_t.wait()
        cp_s.wait()
        gq = pltpu.async_copy(q_hbm.at[tb], qb, sem0)
        gk = pltpu.async_copy(k_hbm.at[sb], kb, sem1)
        gv = pltpu.async_copy(v_hbm.at[sb], vb, sem2)
        gq.wait()
        gk.wait()
        gv.wait()
        pltpu.sync_copy(qb, qg_hbm.at[pl.ds(base, n)])
        pltpu.sync_copy(kb, kg_hbm.at[pl.ds(base, n)])
        pltpu.sync_copy(vb, vg_hbm.at[pl.ds(base, n)])

    def _chunk(ci, carry):
        _do_chunk(tile_base + ci * C, C, tgtbuf, srcbuf, qbuf, kbuf, vbuf)
        return carry

    lax.fori_loop(0, NCH_A, _chunk, 0)
    _do_chunk(tile_base + NCH_A * C, TAIL_A, tgtbt, srcbt, qbt, kbt, vbt)


def _sc_gather(q, k, v, tgt, src):
    mesh = plsc.VectorSubcoreMesh(
        core_axis_name="c", subcore_axis_name="s", num_cores=NC, num_subcores=NS)
    fn = pl.kernel(
        _sc_gather_body,
        out_type=(jax.ShapeDtypeStruct((E, D), _f32),
                  jax.ShapeDtypeStruct((E, D), _f32),
                  jax.ShapeDtypeStruct((E, D), _f32)),
        mesh=mesh,
        scratch_types=[
            pltpu.VMEM((C, D), _f32),
            pltpu.VMEM((C, D), _f32),
            pltpu.VMEM((C, D), _f32),
            pltpu.VMEM((C,), _i32),
            pltpu.VMEM((C,), _i32),
            pltpu.VMEM((TAIL_A, D), _f32),
            pltpu.VMEM((TAIL_A, D), _f32),
            pltpu.VMEM((TAIL_A, D), _f32),
            pltpu.VMEM((TAIL_A,), _i32),
            pltpu.VMEM((TAIL_A,), _i32),
            pltpu.SemaphoreType.DMA,
            pltpu.SemaphoreType.DMA,
            pltpu.SemaphoreType.DMA,
        ],
    )
    return fn(q, k, v, tgt, src)


def _edge_body(qg_ref, kg_ref, vg_ref, w_ref, s_ref):
    p = qg_ref[...] * kg_ref[...]
    colh = lax.broadcasted_iota(_i32, (H, D), 1) // DH
    rowh = lax.broadcasted_iota(_i32, (H, D), 0)
    selm = (colh == rowh).astype(_f32)
    s4 = jnp.dot(p, selm.T, preferred_element_type=_f32)
    exrep = jnp.dot(jnp.exp(s4), selm, preferred_element_type=_f32)
    w_ref[...] = vg_ref[...] * exrep
    s_ref[...] = exrep


def _edge(qg, kg, vg):
    blk = 2000
    row_spec = pl.BlockSpec((blk, D), lambda i: (i, 0))
    return pl.pallas_call(
        _edge_body,
        grid=(E // blk,),
        in_specs=[row_spec, row_spec, row_spec],
        out_specs=[row_spec, row_spec],
        out_shape=[jax.ShapeDtypeStruct((E, D), _f32)] * 2,
    )(qg, kg, vg)


def _sc_scatter_body(w_hbm, s_hbm, tgt_hbm,
                     acc_hbm,
                     buf, tgtbuf, buft, tgtbt,
                     sem0, sem1,
                     acc_sh):
    cid = lax.axis_index("c")
    sid = lax.axis_index("s")
    row0 = sid * RB
    is_last = sid == NS - 1
    tile_base = sid * EPT_B
    zeros16 = jnp.zeros((L,), _f32)

    def _zw(i, c):
        for j in range(D // L):
            buf[i, pl.ds(j * L, L)] = zeros16
        return c
    lax.fori_loop(0, C, _zw, 0)
    off = 0
    while off < RB:
        sz = min(C, RB - off)
        pltpu.sync_copy(buf.at[pl.ds(0, sz)], acc_sh.at[pl.ds(row0 + off, sz)])
        off += sz

    @pl.when(is_last)
    def _zero_tail():
        pltpu.sync_copy(buf.at[pl.ds(0, RTAIL)], acc_sh.at[pl.ds(RB * NS, RTAIL)])

    plsc.subcore_barrier()

    def _do_chunk(rows_hbm, base, n, tb, rb):
        cp_t = pltpu.async_copy(tgt_hbm.at[pl.ds(base, n)], tb, sem0)
        cp_r = pltpu.async_copy(rows_hbm.at[pl.ds(base, n)], rb, sem1)
        cp_t.wait()
        cp_r.wait()
        pltpu.sync_copy(rb, acc_sh.at[tb], add=True)

    def _mk_chunk(rows_hbm):
        def _chunk(ci, carry):
            _do_chunk(rows_hbm, tile_base + ci * C, C, tgtbuf, buf)
            return carry
        return _chunk

    @pl.when(cid == 0)
    def _do_w():
        lax.fori_loop(0, NCH_B, _mk_chunk(w_hbm), 0)
        _do_chunk(w_hbm, tile_base + NCH_B * C, TAIL_B, tgtbt, buft)

    @pl.when(cid == 1)
    def _do_s():
        lax.fori_loop(0, NCH_B, _mk_chunk(s_hbm), 0)
        _do_chunk(s_hbm, tile_base + NCH_B * C, TAIL_B, tgtbt, buft)

    plsc.subcore_barrier()

    pltpu.sync_copy(acc_sh.at[pl.ds(row0, RB)], acc_hbm.at[cid, pl.ds(row0, RB)])

    @pl.when(is_last)
    def _write_tail():
        pltpu.sync_copy(acc_sh.at[pl.ds(RB * NS, RTAIL)],
                        acc_hbm.at[cid, pl.ds(RB * NS, RTAIL)])


def _sc_scatter(w, s, tgt):
    mesh = plsc.VectorSubcoreMesh(
        core_axis_name="c", subcore_axis_name="s", num_cores=NC, num_subcores=NS)
    fn = pl.kernel(
        _sc_scatter_body,
        out_type=jax.ShapeDtypeStruct((NC, N, D), _f32),
        mesh=mesh,
        scratch_types=[
            pltpu.VMEM((C, D), _f32),
            pltpu.VMEM((C,), _i32),
            pltpu.VMEM((TAIL_B, D), _f32),
            pltpu.VMEM((TAIL_B,), _i32),
            pltpu.SemaphoreType.DMA,
            pltpu.SemaphoreType.DMA,
            pltpu.VMEM_SHARED((N, D), _f32),
        ],
    )
    return fn(w, s, tgt)


def _final_body(x_ref, acc_ref, wo_ref, bo_ref, g_ref, b_ref, o_ref):
    w = acc_ref[0] / (acc_ref[1] + 1e-10)
    y = jnp.dot(w, wo_ref[...], preferred_element_type=_f32) + bo_ref[...] + x_ref[...]
    mu = jnp.mean(y, axis=1, keepdims=True)
    dev = y - mu
    var = jnp.mean(dev * dev, axis=1, keepdims=True)
    o_ref[...] = g_ref[...] * (dev * lax.rsqrt(var + 1e-5)) + b_ref[...]


def _final(x, acc, wo, bo, gamma, beta):
    blk = 1000
    row_spec = pl.BlockSpec((blk, D), lambda i: (i, 0))
    return pl.pallas_call(
        _final_body,
        grid=(N // blk,),
        in_specs=[
            row_spec,
            pl.BlockSpec((NC, blk, D), lambda i: (0, i, 0)),
            pl.BlockSpec((D, D), lambda i: (0, 0)),
            pl.BlockSpec((1, D), lambda i: (0, 0)),
            pl.BlockSpec((1, D), lambda i: (0, 0)),
            pl.BlockSpec((1, D), lambda i: (0, 0)),
        ],
        out_specs=row_spec,
        out_shape=jax.ShapeDtypeStruct((N, D), _f32),
    )(x, acc, wo, bo.reshape(1, D), gamma.reshape(1, D), beta.reshape(1, D))


@jax.jit
def kernel(node_features, edge_index, Wq, bq, Wk, bk, Wv, bv, Wo, bo, gamma, beta):
    src = edge_index[0].astype(_i32)
    tgt = edge_index[1].astype(_i32)
    q, k, v = _qkv(node_features, Wq, bq, Wk, bk, Wv, bv)
    qg, kg, vg = _sc_gather(q, k, v, tgt, src)
    w, s = _edge(qg, kg, vg)
    acc = _sc_scatter(w, s, tgt)
    return _final(node_features, acc, Wo, bo, gamma, beta)


# speedup vs baseline: 1.1654x; 1.0653x over previous
import math

import jax
import jax.numpy as jnp
from jax import lax
from jax.experimental import pallas as pl
from jax.experimental.pallas import tpu as pltpu
from jax.experimental.pallas import tpu_sc as plsc

D = 128
H = 4
DH = D // H
N = 10000
E = 320000
NC = 2
NS = 16
NW = NC * NS
C = 128
EPT_A = E // NW
NCH_A = EPT_A // C
TAIL_A = EPT_A - NCH_A * C
EPT_B = E // NS
NCH_B = EPT_B // C
TAIL_B = EPT_B - NCH_B * C
RB = 624
RTAIL = N - RB * NS
L = 16

_f32 = jnp.float32
_i32 = jnp.int32


def _qkv_body(x_ref, wq_ref, bq_ref, wk_ref, bk_ref, wv_ref, bv_ref,
              q_ref, k_ref, v_ref):
    x = x_ref[...]
    q_ref[...] = jnp.dot(x, wq_ref[...], preferred_element_type=_f32) + bq_ref[...]
    k = jnp.dot(x, wk_ref[...], preferred_element_type=_f32) + bk_ref[...]
    k_ref[...] = k * (1.0 / math.sqrt(DH))
    v_ref[...] = jnp.dot(x, wv_ref[...], preferred_element_type=_f32) + bv_ref[...]


def _qkv(x, wq, bq, wk, bk, wv, bv):
    blk = 1000
    row_spec = pl.BlockSpec((blk, D), lambda i: (i, 0))
    full_w = pl.BlockSpec((D, D), lambda i: (0, 0))
    full_b = pl.BlockSpec((1, D), lambda i: (0, 0))
    return pl.pallas_call(
        _qkv_body,
        grid=(N // blk,),
        in_specs=[row_spec, full_w, full_b, full_w, full_b, full_w, full_b],
        out_specs=[row_spec, row_spec, row_spec],
        out_shape=[jax.ShapeDtypeStruct((N, D), _f32)] * 3,
    )(x, wq, bq.reshape(1, D), wk, bk.reshape(1, D), wv, bv.reshape(1, D))


def _sc_gather_body(q_hbm, k_hbm, v_hbm, tgt_hbm, src_hbm,
                    qg_hbm, kg_hbm, vg_hbm,
                    qb0, kb0, vb0, tb0, sb0,
                    qb1, kb1, vb1, tb1, sb1,
                    qbt, kbt, vbt, tgtbt, srcbt,
                    s0, s1, s2, s3, s4, s5):
    cid = lax.axis_index("c")
    sid = lax.axis_index("s")
    wid = sid * NC + cid
    tile_base = wid * EPT_A

    def _writeout(base, n, qb, kb, vb):
        pltpu.sync_copy(qb, qg_hbm.at[pl.ds(base, n)])
        pltpu.sync_copy(kb, kg_hbm.at[pl.ds(base, n)])
        pltpu.sync_copy(vb, vg_hbm.at[pl.ds(base, n)])

    def _pair(ci, carry):
        base_a = tile_base + (2 * ci) * C
        base_b = base_a + C
        ta = pltpu.async_copy(tgt_hbm.at[pl.ds(base_a, C)], tb0, s0)
        sa = pltpu.async_copy(src_hbm.at[pl.ds(base_a, C)], sb0, s1)
        tb_ = pltpu.async_copy(tgt_hbm.at[pl.ds(base_b, C)], tb1, s3)
        sb_ = pltpu.async_copy(src_hbm.at[pl.ds(base_b, C)], sb1, s4)
        ta.wait()
        sa.wait()
        gqa = pltpu.async_copy(q_hbm.at[tb0], qb0, s0)
        gka = pltpu.async_copy(k_hbm.at[sb0], kb0, s1)
        gva = pltpu.async_copy(v_hbm.at[sb0], vb0, s2)
        tb_.wait()
        sb_.wait()
        gqb = pltpu.async_copy(q_hbm.at[tb1], qb1, s3)
        gkb = pltpu.async_copy(k_hbm.at[sb1], kb1, s4)
        gvb = pltpu.async_copy(v_hbm.at[sb1], vb1, s5)
        gqa.wait()
        gka.wait()
        gva.wait()
        _writeout(base_a, C, qb0, kb0, vb0)
        gqb.wait()
        gkb.wait()
        gvb.wait()
        _writeout(base_b, C, qb1, kb1, vb1)
        return carry

    lax.fori_loop(0, NCH_A // 2, _pair, 0)

    base_t = tile_base + NCH_A * C
    ct = pltpu.async_copy(tgt_hbm.at[pl.ds(base_t, TAIL_A)], tgtbt, s0)
    cs = pltpu.async_copy(src_hbm.at[pl.ds(base_t, TAIL_A)], srcbt, s1)
    ct.wait()
    cs.wait()
    gq = pltpu.async_copy(q_hbm.at[tgtbt], qbt, s0)
    gk = pltpu.async_copy(k_hbm.at[srcbt], kbt, s1)
    gv = pltpu.async_copy(v_hbm.at[srcbt], vbt, s2)
    gq.wait()
    gk.wait()
    gv.wait()
    _writeout(base_t, TAIL_A, qbt, kbt, vbt)


def _sc_gather(q, k, v, tgt, src):
    mesh = plsc.VectorSubcoreMesh(
        core_axis_name="c", subcore_axis_name="s", num_cores=NC, num_subcores=NS)
    fn = pl.kernel(
        _sc_gather_body,
        out_type=(jax.ShapeDtypeStruct((E, D), _f32),
                  jax.ShapeDtypeStruct((E, D), _f32),
                  jax.ShapeDtypeStruct((E, D), _f32)),
        mesh=mesh,
        scratch_types=[
            pltpu.VMEM((C, D), _f32),
            pltpu.VMEM((C, D), _f32),
            pltpu.VMEM((C, D), _f32),
            pltpu.VMEM((C,), _i32),
            pltpu.VMEM((C,), _i32),
            pltpu.VMEM((C, D), _f32),
            pltpu.VMEM((C, D), _f32),
            pltpu.VMEM((C, D), _f32),
            pltpu.VMEM((C,), _i32),
            pltpu.VMEM((C,), _i32),
            pltpu.VMEM((TAIL_A, D), _f32),
            pltpu.VMEM((TAIL_A, D), _f32),
            pltpu.VMEM((TAIL_A, D), _f32),
            pltpu.VMEM((TAIL_A,), _i32),
            pltpu.VMEM((TAIL_A,), _i32),
            pltpu.SemaphoreType.DMA,
            pltpu.SemaphoreType.DMA,
            pltpu.SemaphoreType.DMA,
            pltpu.SemaphoreType.DMA,
            pltpu.SemaphoreType.DMA,
            pltpu.SemaphoreType.DMA,
        ],
    )
    return fn(q, k, v, tgt, src)


def _edge_body(qg_ref, kg_ref, vg_ref, w_ref, s_ref):
    p = qg_ref[...] * kg_ref[...]
    colh = lax.broadcasted_iota(_i32, (H, D), 1) // DH
    rowh = lax.broadcasted_iota(_i32, (H, D), 0)
    selm = (colh == rowh).astype(_f32)
    s4 = jnp.dot(p, selm.T, preferred_element_type=_f32)
    exrep = jnp.dot(jnp.exp(s4), selm, preferred_element_type=_f32)
    w_ref[...] = vg_ref[...] * exrep
    s_ref[...] = exrep


def _edge(qg, kg, vg):
    blk = 2000
    row_spec = pl.BlockSpec((blk, D), lambda i: (i, 0))
    return pl.pallas_call(
        _edge_body,
        grid=(E // blk,),
        in_specs=[row_spec, row_spec, row_spec],
        out_specs=[row_spec, row_spec],
        out_shape=[jax.ShapeDtypeStruct((E, D), _f32)] * 2,
    )(qg, kg, vg)


def _sc_scatter_body(w_hbm, s_hbm, tgt_hbm,
                     acc_hbm,
                     buf, tgtbuf, buft, tgtbt,
                     sem0, sem1,
                     acc_sh):
    cid = lax.axis_index("c")
    sid = lax.axis_index("s")
    row0 = sid * RB
    is_last = sid == NS - 1
    tile_base = sid * EPT_B
    zeros16 = jnp.zeros((L,), _f32)

    def _zw(i, c):
        for j in range(D // L):
            buf[i, pl.ds(j * L, L)] = zeros16
        return c
    lax.fori_loop(0, C, _zw, 0)
    off = 0
    while off < RB:
        sz = min(C, RB - off)
        pltpu.sync_copy(buf.at[pl.ds(0, sz)], acc_sh.at[pl.ds(row0 + off, sz)])
        off += sz

    @pl.when(is_last)
    def _zero_tail():
        pltpu.sync_copy(buf.at[pl.ds(0, RTAIL)], acc_sh.at[pl.ds(RB * NS, RTAIL)])

    plsc.subcore_barrier()

    def _do_chunk(rows_hbm, base, n, tb, rb):
        cp_t = pltpu.async_copy(tgt_hbm.at[pl.ds(base, n)], tb, sem0)
        cp_r = pltpu.async_copy(rows_hbm.at[pl.ds(base, n)], rb, sem1)
        cp_t.wait()
        cp_r.wait()
        pltpu.sync_copy(rb, acc_sh.at[tb], add=True)

    def _mk_chunk(rows_hbm):
        def _chunk(ci, carry):
            _do_chunk(rows_hbm, tile_base + ci * C, C, tgtbuf, buf)
            return carry
        return _chunk

    @pl.when(cid == 0)
    def _do_w():
        lax.fori_loop(0, NCH_B, _mk_chunk(w_hbm), 0)
        _do_chunk(w_hbm, tile_base + NCH_B * C, TAIL_B, tgtbt, buft)

    @pl.when(cid == 1)
    def _do_s():
        lax.fori_loop(0, NCH_B, _mk_chunk(s_hbm), 0)
        _do_chunk(s_hbm, tile_base + NCH_B * C, TAIL_B, tgtbt, buft)

    plsc.subcore_barrier()

    pltpu.sync_copy(acc_sh.at[pl.ds(row0, RB)], acc_hbm.at[cid, pl.ds(row0, RB)])

    @pl.when(is_last)
    def _write_tail():
        pltpu.sync_copy(acc_sh.at[pl.ds(RB * NS, RTAIL)],
                        acc_hbm.at[cid, pl.ds(RB * NS, RTAIL)])


def _sc_scatter(w, s, tgt):
    mesh = plsc.VectorSubcoreMesh(
        core_axis_name="c", subcore_axis_name="s", num_cores=NC, num_subcores=NS)
    fn = pl.kernel(
        _sc_scatter_body,
        out_type=jax.ShapeDtypeStruct((NC, N, D), _f32),
        mesh=mesh,
        scratch_types=[
            pltpu.VMEM((C, D), _f32),
            pltpu.VMEM((C,), _i32),
            pltpu.VMEM((TAIL_B, D), _f32),
            pltpu.VMEM((TAIL_B,), _i32),
            pltpu.SemaphoreType.DMA,
            pltpu.SemaphoreType.DMA,
            pltpu.VMEM_SHARED((N, D), _f32),
        ],
    )
    return fn(w, s, tgt)


def _final_body(x_ref, acc_ref, wo_ref, bo_ref, g_ref, b_ref, o_ref):
    w = acc_ref[0] / (acc_ref[1] + 1e-10)
    y = jnp.dot(w, wo_ref[...], preferred_element_type=_f32) + bo_ref[...] + x_ref[...]
    mu = jnp.mean(y, axis=1, keepdims=True)
    dev = y - mu
    var = jnp.mean(dev * dev, axis=1, keepdims=True)
    o_ref[...] = g_ref[...] * (dev * lax.rsqrt(var + 1e-5)) + b_ref[...]


def _final(x, acc, wo, bo, gamma, beta):
    blk = 1000
    row_spec = pl.BlockSpec((blk, D), lambda i: (i, 0))
    return pl.pallas_call(
        _final_body,
        grid=(N // blk,),
        in_specs=[
            row_spec,
            pl.BlockSpec((NC, blk, D), lambda i: (0, i, 0)),
            pl.BlockSpec((D, D), lambda i: (0, 0)),
            pl.BlockSpec((1, D), lambda i: (0, 0)),
            pl.BlockSpec((1, D), lambda i: (0, 0)),
            pl.BlockSpec((1, D), lambda i: (0, 0)),
        ],
        out_specs=row_spec,
        out_shape=jax.ShapeDtypeStruct((N, D), _f32),
    )(x, acc, wo, bo.reshape(1, D), gamma.reshape(1, D), beta.reshape(1, D))


@jax.jit
def kernel(node_features, edge_index, Wq, bq, Wk, bk, Wv, bv, Wo, bo, gamma, beta):
    src = edge_index[0].astype(_i32)
    tgt = edge_index[1].astype(_i32)
    q, k, v = _qkv(node_features, Wq, bq, Wk, bk, Wv, bv)
    qg, kg, vg = _sc_gather(q, k, v, tgt, src)
    w, s = _edge(qg, kg, vg)
    acc = _sc_scatter(w, s, tgt)
    return _final(node_features, acc, Wo, bo, gamma, beta)


# speedup vs baseline: 1.2169x; 1.0442x over previous
import math

import jax
import jax.numpy as jnp
from jax import lax
from jax.experimental import pallas as pl
from jax.experimental.pallas import tpu as pltpu
from jax.experimental.pallas import tpu_sc as plsc

D = 128
H = 4
DH = D // H
N = 10000
E = 320000
NC = 2
NS = 16
NW = NC * NS
C = 128
EPT_A = E // NW
NCH_A = EPT_A // C
TAIL_A = EPT_A - NCH_A * C
EPT_B = E // NS
NCH_B = EPT_B // C
TAIL_B = EPT_B - NCH_B * C
RB = 624
RTAIL = N - RB * NS
L = 16

_f32 = jnp.float32
_i32 = jnp.int32


def _qkv_body(x_ref, wq_ref, bq_ref, wk_ref, bk_ref, wv_ref, bv_ref,
              q_ref, k_ref, v_ref):
    x = x_ref[...]
    q_ref[...] = jnp.dot(x, wq_ref[...], preferred_element_type=_f32) + bq_ref[...]
    k = jnp.dot(x, wk_ref[...], preferred_element_type=_f32) + bk_ref[...]
    k_ref[...] = k * (1.0 / math.sqrt(DH))
    v_ref[...] = jnp.dot(x, wv_ref[...], preferred_element_type=_f32) + bv_ref[...]


def _qkv(x, wq, bq, wk, bk, wv, bv):
    blk = 1000
    row_spec = pl.BlockSpec((blk, D), lambda i: (i, 0))
    full_w = pl.BlockSpec((D, D), lambda i: (0, 0))
    full_b = pl.BlockSpec((1, D), lambda i: (0, 0))
    return pl.pallas_call(
        _qkv_body,
        grid=(N // blk,),
        in_specs=[row_spec, full_w, full_b, full_w, full_b, full_w, full_b],
        out_specs=[row_spec, row_spec, row_spec],
        out_shape=[jax.ShapeDtypeStruct((N, D), _f32)] * 3,
    )(x, wq, bq.reshape(1, D), wk, bk.reshape(1, D), wv, bv.reshape(1, D))


def _sc_gather_body(q_hbm, k_hbm, v_hbm, tgt_hbm, src_hbm,
                    qg_hbm, kg_hbm, vg_hbm,
                    qb0, kb0, vb0, tb0, sb0,
                    qb1, kb1, vb1, tb1, sb1,
                    qbt, kbt, vbt, tgtbt, srcbt,
                    s0, s1, s2, s3, s4, s5):
    cid = lax.axis_index("c")
    sid = lax.axis_index("s")
    wid = sid * NC + cid
    tile_base = wid * EPT_A

    def _writeout(base, n, qb, kb, vb):
        pltpu.sync_copy(qb, qg_hbm.at[pl.ds(base, n)])
        pltpu.sync_copy(kb, kg_hbm.at[pl.ds(base, n)])
        pltpu.sync_copy(vb, vg_hbm.at[pl.ds(base, n)])

    def _pair(ci, carry):
        base_a = tile_base + (2 * ci) * C
        base_b = base_a + C
        ta = pltpu.async_copy(tgt_hbm.at[pl.ds(base_a, C)], tb0, s0)
        sa = pltpu.async_copy(src_hbm.at[pl.ds(base_a, C)], sb0, s1)
        tb_ = pltpu.async_copy(tgt_hbm.at[pl.ds(base_b, C)], tb1, s3)
        sb_ = pltpu.async_copy(src_hbm.at[pl.ds(base_b, C)], sb1, s4)
        ta.wait()
        sa.wait()
        gqa = pltpu.async_copy(q_hbm.at[tb0], qb0, s0)
        gka = pltpu.async_copy(k_hbm.at[sb0], kb0, s1)
        gva = pltpu.async_copy(v_hbm.at[sb0], vb0, s2)
        tb_.wait()
        sb_.wait()
        gqb = pltpu.async_copy(q_hbm.at[tb1], qb1, s3)
        gkb = pltpu.async_copy(k_hbm.at[sb1], kb1, s4)
        gvb = pltpu.async_copy(v_hbm.at[sb1], vb1, s5)
        gqa.wait()
        gka.wait()
        gva.wait()
        _writeout(base_a, C, qb0, kb0, vb0)
        gqb.wait()
        gkb.wait()
        gvb.wait()
        _writeout(base_b, C, qb1, kb1, vb1)
        return carry

    lax.fori_loop(0, NCH_A // 2, _pair, 0)

    base_t = tile_base + NCH_A * C
    ct = pltpu.async_copy(tgt_hbm.at[pl.ds(base_t, TAIL_A)], tgtbt, s0)
    cs = pltpu.async_copy(src_hbm.at[pl.ds(base_t, TAIL_A)], srcbt, s1)
    ct.wait()
    cs.wait()
    gq = pltpu.async_copy(q_hbm.at[tgtbt], qbt, s0)
    gk = pltpu.async_copy(k_hbm.at[srcbt], kbt, s1)
    gv = pltpu.async_copy(v_hbm.at[srcbt], vbt, s2)
    gq.wait()
    gk.wait()
    gv.wait()
    _writeout(base_t, TAIL_A, qbt, kbt, vbt)


def _sc_gather(q, k, v, tgt, src):
    mesh = plsc.VectorSubcoreMesh(
        core_axis_name="c", subcore_axis_name="s", num_cores=NC, num_subcores=NS)
    fn = pl.kernel(
        _sc_gather_body,
        out_type=(jax.ShapeDtypeStruct((E, D), _f32),
                  jax.ShapeDtypeStruct((E, D), _f32),
                  jax.ShapeDtypeStruct((E, D), _f32)),
        mesh=mesh,
        scratch_types=[
            pltpu.VMEM((C, D), _f32),
            pltpu.VMEM((C, D), _f32),
            pltpu.VMEM((C, D), _f32),
            pltpu.VMEM((C,), _i32),
            pltpu.VMEM((C,), _i32),
            pltpu.VMEM((C, D), _f32),
            pltpu.VMEM((C, D), _f32),
            pltpu.VMEM((C, D), _f32),
            pltpu.VMEM((C,), _i32),
            pltpu.VMEM((C,), _i32),
            pltpu.VMEM((TAIL_A, D), _f32),
            pltpu.VMEM((TAIL_A, D), _f32),
            pltpu.VMEM((TAIL_A, D), _f32),
            pltpu.VMEM((TAIL_A,), _i32),
            pltpu.VMEM((TAIL_A,), _i32),
            pltpu.SemaphoreType.DMA,
            pltpu.SemaphoreType.DMA,
            pltpu.SemaphoreType.DMA,
            pltpu.SemaphoreType.DMA,
            pltpu.SemaphoreType.DMA,
            pltpu.SemaphoreType.DMA,
        ],
    )
    return fn(q, k, v, tgt, src)


def _edge_body(qg_ref, kg_ref, vg_ref, w_ref, s_ref):
    p = qg_ref[...] * kg_ref[...]
    colh = lax.broadcasted_iota(_i32, (H, D), 1) // DH
    rowh = lax.broadcasted_iota(_i32, (H, D), 0)
    selm = (colh == rowh).astype(_f32)
    s4 = jnp.dot(p, selm.T, preferred_element_type=_f32)
    exrep = jnp.dot(jnp.exp(s4), selm, preferred_element_type=_f32)
    w_ref[...] = vg_ref[...] * exrep
    s_ref[...] = exrep


def _edge(qg, kg, vg):
    blk = 2000
    row_spec = pl.BlockSpec((blk, D), lambda i: (i, 0))
    return pl.pallas_call(
        _edge_body,
        grid=(E // blk,),
        in_specs=[row_spec, row_spec, row_spec],
        out_specs=[row_spec, row_spec],
        out_shape=[jax.ShapeDtypeStruct((E, D), _f32)] * 2,
    )(qg, kg, vg)


def _sc_scatter_body(w_hbm, s_hbm, tgt_hbm,
                     acc_hbm,
                     buf, tgtbuf, buf1, tgtbuf1, buft, tgtbt,
                     sem0, sem1, sem2, sem3,
                     acc_sh):
    cid = lax.axis_index("c")
    sid = lax.axis_index("s")
    row0 = sid * RB
    is_last = sid == NS - 1
    tile_base = sid * EPT_B
    zeros16 = jnp.zeros((L,), _f32)

    def _zw(i, c):
        for j in range(D // L):
            buf[i, pl.ds(j * L, L)] = zeros16
        return c
    lax.fori_loop(0, C, _zw, 0)
    off = 0
    while off < RB:
        sz = min(C, RB - off)
        pltpu.sync_copy(buf.at[pl.ds(0, sz)], acc_sh.at[pl.ds(row0 + off, sz)])
        off += sz

    @pl.when(is_last)
    def _zero_tail():
        pltpu.sync_copy(buf.at[pl.ds(0, RTAIL)], acc_sh.at[pl.ds(RB * NS, RTAIL)])

    plsc.subcore_barrier()

    def _do_chunk(rows_hbm, base, n, tb, rb):
        cp_t = pltpu.async_copy(tgt_hbm.at[pl.ds(base, n)], tb, sem0)
        cp_r = pltpu.async_copy(rows_hbm.at[pl.ds(base, n)], rb, sem1)
        cp_t.wait()
        cp_r.wait()
        pltpu.sync_copy(rb, acc_sh.at[tb], add=True)

    def _mk_pair(rows_hbm):
        def _pair(ci, carry):
            base_a = tile_base + (2 * ci) * C
            base_b = base_a + C
            ta = pltpu.async_copy(tgt_hbm.at[pl.ds(base_a, C)], tgtbuf, sem0)
            ra = pltpu.async_copy(rows_hbm.at[pl.ds(base_a, C)], buf, sem1)
            tb_ = pltpu.async_copy(tgt_hbm.at[pl.ds(base_b, C)], tgtbuf1, sem2)
            rb_ = pltpu.async_copy(rows_hbm.at[pl.ds(base_b, C)], buf1, sem3)
            ta.wait()
            ra.wait()
            pltpu.sync_copy(buf, acc_sh.at[tgtbuf], add=True)
            tb_.wait()
            rb_.wait()
            pltpu.sync_copy(buf1, acc_sh.at[tgtbuf1], add=True)
            return carry
        return _pair

    @pl.when(cid == 0)
    def _do_w():
        lax.fori_loop(0, NCH_B // 2, _mk_pair(w_hbm), 0)
        _do_chunk(w_hbm, tile_base + NCH_B * C, TAIL_B, tgtbt, buft)

    @pl.when(cid == 1)
    def _do_s():
        lax.fori_loop(0, NCH_B // 2, _mk_pair(s_hbm), 0)
        _do_chunk(s_hbm, tile_base + NCH_B * C, TAIL_B, tgtbt, buft)

    plsc.subcore_barrier()

    pltpu.sync_copy(acc_sh.at[pl.ds(row0, RB)], acc_hbm.at[cid, pl.ds(row0, RB)])

    @pl.when(is_last)
    def _write_tail():
        pltpu.sync_copy(acc_sh.at[pl.ds(RB * NS, RTAIL)],
                        acc_hbm.at[cid, pl.ds(RB * NS, RTAIL)])


def _sc_scatter(w, s, tgt):
    mesh = plsc.VectorSubcoreMesh(
        core_axis_name="c", subcore_axis_name="s", num_cores=NC, num_subcores=NS)
    fn = pl.kernel(
        _sc_scatter_body,
        out_type=jax.ShapeDtypeStruct((NC, N, D), _f32),
        mesh=mesh,
        scratch_types=[
            pltpu.VMEM((C, D), _f32),
            pltpu.VMEM((C,), _i32),
            pltpu.VMEM((C, D), _f32),
            pltpu.VMEM((C,), _i32),
            pltpu.VMEM((TAIL_B, D), _f32),
            pltpu.VMEM((TAIL_B,), _i32),
            pltpu.SemaphoreType.DMA,
            pltpu.SemaphoreType.DMA,
            pltpu.SemaphoreType.DMA,
            pltpu.SemaphoreType.DMA,
            pltpu.VMEM_SHARED((N, D), _f32),
        ],
    )
    return fn(w, s, tgt)


def _final_body(x_ref, acc_ref, wo_ref, bo_ref, g_ref, b_ref, o_ref):
    w = acc_ref[0] / (acc_ref[1] + 1e-10)
    y = jnp.dot(w, wo_ref[...], preferred_element_type=_f32) + bo_ref[...] + x_ref[...]
    mu = jnp.mean(y, axis=1, keepdims=True)
    dev = y - mu
    var = jnp.mean(dev * dev, axis=1, keepdims=True)
    o_ref[...] = g_ref[...] * (dev * lax.rsqrt(var + 1e-5)) + b_ref[...]


def _final(x, acc, wo, bo, gamma, beta):
    blk = 1000
    row_spec = pl.BlockSpec((blk, D), lambda i: (i, 0))
    return pl.pallas_call(
        _final_body,
        grid=(N // blk,),
        in_specs=[
            row_spec,
            pl.BlockSpec((NC, blk, D), lambda i: (0, i, 0)),
            pl.BlockSpec((D, D), lambda i: (0, 0)),
            pl.BlockSpec((1, D), lambda i: (0, 0)),
            pl.BlockSpec((1, D), lambda i: (0, 0)),
            pl.BlockSpec((1, D), lambda i: (0, 0)),
        ],
        out_specs=row_spec,
        out_shape=jax.ShapeDtypeStruct((N, D), _f32),
    )(x, acc, wo, bo.reshape(1, D), gamma.reshape(1, D), beta.reshape(1, D))


@jax.jit
def kernel(node_features, edge_index, Wq, bq, Wk, bk, Wv, bv, Wo, bo, gamma, beta):
    src = edge_index[0].astype(_i32)
    tgt = edge_index[1].astype(_i32)
    q, k, v = _qkv(node_features, Wq, bq, Wk, bk, Wv, bv)
    qg, kg, vg = _sc_gather(q, k, v, tgt, src)
    w, s = _edge(qg, kg, vg)
    acc = _sc_scatter(w, s, tgt)
    return _final(node_features, acc, Wo, bo, gamma, beta)
